# baseline (speedup 1.0000x reference)
import functools

import jax
import jax.numpy as jnp
from jax import lax
from jax.experimental import pallas as pl
from jax.experimental.pallas import tpu as pltpu
from jax.experimental.pallas import tpu_sc as plsc

BATCH = 4096
FIELDS = 26
EMBED_DIM = 32
NC = 2
NS = 16
NW = NC * NS
SPT = BATCH // NW
ROWS = SPT * FIELDS
NCHUNK = FIELDS

_mesh = plsc.VectorSubcoreMesh(core_axis_name="c", subcore_axis_name="s")


@functools.partial(
    pl.kernel,
    out_type=jax.ShapeDtypeStruct((BATCH,), jnp.float32),
    mesh=_mesh,
    scratch_types=[
        pltpu.VMEM((NCHUNK, SPT), jnp.int32),
        pltpu.VMEM((ROWS, EMBED_DIM), jnp.float32),
        pltpu.VMEM((ROWS,), jnp.float32),
        pltpu.VMEM((16 * SPT,), jnp.float32),
        pltpu.VMEM((SPT,), jnp.float32),
        pltpu.VMEM((16,), jnp.float32),
        pltpu.SemaphoreType.DMA,
    ],
    compiler_params=pltpu.CompilerParams(needs_layout_passes=False,
                                         use_tc_tiling_on_sc=False),
)
def _fm_sc(xr, lin, bias16, emb, out, idx_v, rows_v, lin_v, comb_v, out_v,
           bias_v, sem):
    wid = lax.axis_index("s") * NC + lax.axis_index("c")

    pltpu.sync_copy(xr.at[wid], idx_v)
    pltpu.sync_copy(bias16, bias_v)

    def fire(j, carry):
        pltpu.async_copy(emb.at[idx_v.at[j]], rows_v.at[pl.ds(j * SPT, SPT)], sem)
        pltpu.async_copy(lin.at[idx_v.at[j]], lin_v.at[pl.ds(j * SPT, SPT)], sem)
        return carry

    lax.fori_loop(0, NCHUNK, fire, 0)
    pltpu.make_async_copy(emb.at[pl.ds(0, ROWS)], rows_v, sem).wait()
    pltpu.make_async_copy(lin.at[pl.ds(0, ROWS)], lin_v, sem).wait()

    mask10 = jnp.where(lax.iota(jnp.int32, 16) < FIELDS - 16, 1.0, 0.0)
    iota16 = lax.iota(jnp.int32, 16)

    def sample(s, carry):
        r0 = s * FIELDS
        acc_lo = rows_v[r0, pl.ds(0, 16)]
        acc_hi = rows_v[r0, pl.ds(16, 16)]
        acc_sq = acc_lo * acc_lo + acc_hi * acc_hi
        for f in range(1, FIELDS):
            lo = rows_v[r0 + f, pl.ds(0, 16)]
            hi = rows_v[r0 + f, pl.ds(16, 16)]
            acc_lo = acc_lo + lo
            acc_hi = acc_hi + hi
            acc_sq = acc_sq + lo * lo + hi * hi
        l1 = lin_v[pl.ds(r0, 16)]
        l2 = lin_v[pl.ds(r0 + 16, 16)]
        combo = 0.5 * (acc_lo * acc_lo + acc_hi * acc_hi - acc_sq)
        combo = combo + l1 + l2 * mask10
        plsc.store_scatter(comb_v, [iota16 * SPT + s], combo)
        return carry

    lax.fori_loop(0, SPT, sample, 0)

    bias_vec = bias_v[pl.ds(0, 16)]
    for c in range(SPT // 16):
        z = comb_v[pl.ds(c * 16, 16)]
        for k in range(1, 16):
            z = z + comb_v[pl.ds(k * SPT + c * 16, 16)]
        z = z + bias_vec
        out_v[pl.ds(c * 16, 16)] = 1.0 / (1.0 + jnp.exp(-z))

    pltpu.sync_copy(out_v, out.at[pl.ds(wid * SPT, SPT)])


def kernel(x, linear_w, bias, embedding_w):
    xr = x.astype(jnp.int32).reshape(NW, NCHUNK, SPT)
    lin = linear_w.reshape(-1)
    bias16 = jnp.broadcast_to(bias.astype(jnp.float32), (16,))
    out = _fm_sc(xr, lin, bias16, embedding_w)
    return out.reshape(BATCH, 1)
